# baseline (speedup 1.0000x reference)
import functools

import jax
import jax.numpy as jnp
from jax import lax
from jax.experimental import pallas as pl
from jax.experimental.pallas import tpu as pltpu
from jax.experimental.pallas import tpu_sc as plsc

N = 10000
E = 320000
D = 128
G = 64

NC = 2
NS = 16
NW = NC * NS
CH = 128
R = 80
EP = NW * R * CH
ROWS_PER_TILE = 632
NPAD = NS * ROWS_PER_TILE

_mesh = plsc.VectorSubcoreMesh(core_axis_name="c", subcore_axis_name="s")


def _sc_degree_body(dst_hbm, zero_hbm, ones_hbm, out_hbm, acc, di, ones, sem):
    cid = lax.axis_index("c")
    sid = lax.axis_index("s")
    wid = sid * NC + cid
    r0 = sid * ROWS_PER_TILE
    pltpu.sync_copy(zero_hbm.at[pl.ds(r0, ROWS_PER_TILE)],
                    acc.at[pl.ds(r0, ROWS_PER_TILE)])
    pltpu.sync_copy(ones_hbm, ones)
    pltpu.async_copy(dst_hbm.at[wid], di, sem).wait()
    plsc.subcore_barrier()

    @pl.loop(0, R)
    def _(j):
        pltpu.sync_copy(ones, acc.at[di.at[j]], add=True)

    plsc.subcore_barrier()
    pltpu.sync_copy(acc.at[pl.ds(r0, ROWS_PER_TILE)],
                    out_hbm.at[cid, pl.ds(r0, ROWS_PER_TILE)])


_sc_degree = pl.kernel(
    _sc_degree_body,
    out_type=jax.ShapeDtypeStruct((NC, NPAD, D), jnp.float32),
    mesh=_mesh,
    scratch_types=[
        pltpu.VMEM_SHARED((NPAD, D), jnp.float32),
        pltpu.VMEM((R, CH), jnp.int32),
        pltpu.VMEM((CH, D), jnp.float32),
        pltpu.SemaphoreType.DMA,
    ],
)


BLK = 16
NBLK = R // BLK


def _sc_scatter_body(g_hbm, src_hbm, dst_hbm, zero_hbm, out_hbm,
                     acc, si0, si1, di0, di1, buf0, buf1,
                     sem0, sem1, sema0, sema1, semi):
    cid = lax.axis_index("c")
    sid = lax.axis_index("s")
    wid = sid * NC + cid
    r0 = sid * ROWS_PER_TILE
    pltpu.sync_copy(zero_hbm.at[pl.ds(r0, ROWS_PER_TILE)],
                    acc.at[pl.ds(r0, ROWS_PER_TILE)])
    sis = (si0, si1)
    dis = (di0, di1)
    pltpu.async_copy(src_hbm.at[wid, pl.ds(0, BLK)], si0, semi)
    pltpu.async_copy(dst_hbm.at[wid, pl.ds(0, BLK)], di0, semi)
    plsc.subcore_barrier()

    for b in range(NBLK):
        si = sis[b % 2]
        di = dis[b % 2]
        base = b * BLK
        pltpu.make_async_copy(src_hbm.at[wid, pl.ds(base, BLK)], si,
                              semi).wait()
        pltpu.make_async_copy(dst_hbm.at[wid, pl.ds(base, BLK)], di,
                              semi).wait()
        if b + 1 < NBLK:
            pltpu.async_copy(src_hbm.at[wid, pl.ds(base + BLK, BLK)],
                             sis[(b + 1) % 2], semi)
            pltpu.async_copy(dst_hbm.at[wid, pl.ds(base + BLK, BLK)],
                             dis[(b + 1) % 2], semi)

        pltpu.async_copy(g_hbm.at[si.at[0]], buf0, sem0)
        pltpu.async_copy(g_hbm.at[si.at[1]], buf1, sem1)

        @pl.loop(0, BLK // 2)
        def _(t):
            j = 2 * t
            pltpu.make_async_copy(g_hbm.at[si.at[j]], buf0, sem0).wait()
            pltpu.async_copy(buf0, acc.at[di.at[j]], sema0, add=True)
            pltpu.make_async_copy(g_hbm.at[si.at[j + 1]], buf1, sem1).wait()
            pltpu.async_copy(buf1, acc.at[di.at[j + 1]], sema1, add=True)

            @pl.when(j + 2 < BLK)
            def _():
                pltpu.make_async_copy(buf0, acc.at[di.at[j]], sema0).wait()
                pltpu.async_copy(g_hbm.at[si.at[j + 2]], buf0, sem0)
                pltpu.make_async_copy(buf1, acc.at[di.at[j + 1]],
                                      sema1).wait()
                pltpu.async_copy(g_hbm.at[si.at[j + 3]], buf1, sem1)

            @pl.when(j + 2 >= BLK)
            def _():
                pltpu.make_async_copy(buf0, acc.at[di.at[j]], sema0).wait()
                pltpu.make_async_copy(buf1, acc.at[di.at[j + 1]],
                                      sema1).wait()

    plsc.subcore_barrier()
    pltpu.sync_copy(acc.at[pl.ds(r0, ROWS_PER_TILE)],
                    out_hbm.at[cid, pl.ds(r0, ROWS_PER_TILE)])


_sc_scatter = pl.kernel(
    _sc_scatter_body,
    out_type=jax.ShapeDtypeStruct((NC, NPAD, D), jnp.float32),
    mesh=_mesh,
    scratch_types=[
        pltpu.VMEM_SHARED((NPAD, D), jnp.float32),
        pltpu.VMEM((BLK, CH), jnp.int32),
        pltpu.VMEM((BLK, CH), jnp.int32),
        pltpu.VMEM((BLK, CH), jnp.int32),
        pltpu.VMEM((BLK, CH), jnp.int32),
        pltpu.VMEM((CH, D), jnp.float32),
        pltpu.VMEM((CH, D), jnp.float32),
        pltpu.SemaphoreType.DMA,
        pltpu.SemaphoreType.DMA,
        pltpu.SemaphoreType.DMA,
        pltpu.SemaphoreType.DMA,
        pltpu.SemaphoreType.DMA,
    ],
)


def _dot(a, b, dims):
    return lax.dot_general(a, b, dims, precision=lax.Precision.HIGHEST,
                           preferred_element_type=jnp.float32)


def _mm_body(x_ref, w_ref, o_ref):
    o_ref[...] = _dot(x_ref[...], w_ref[...], (((1,), (0,)), ((), ())))


_tc_mm = pl.pallas_call(
    _mm_body,
    out_shape=jax.ShapeDtypeStruct((N, D), jnp.float32),
)


def _scale_body(degp_ref, h_ref, g_ref, dinv_ref):
    deg = degp_ref[0, pl.ds(0, N), 0:1] + degp_ref[1, pl.ds(0, N), 0:1] + 1.0
    dinv = lax.rsqrt(deg)
    dinv_ref[...] = dinv
    g_ref[pl.ds(0, N), :] = h_ref[...] * dinv
    g_ref[pl.ds(N, NPAD - N), :] = jnp.zeros((NPAD - N, D), jnp.float32)


_tc_scale = pl.pallas_call(
    _scale_body,
    out_shape=(jax.ShapeDtypeStruct((NPAD, D), jnp.float32),
               jax.ShapeDtypeStruct((N, 1), jnp.float32)),
)


def _mid_body(sp_ref, g_ref, dinv_ref, b_ref, w_ref, gout_ref):
    dinv = dinv_ref[...]
    s = (sp_ref[0, pl.ds(0, N), :] + sp_ref[1, pl.ds(0, N), :]
         + g_ref[pl.ds(0, N), :]) * dinv + b_ref[...]
    z = jnp.maximum(s, 0.0)
    h2 = _dot(z, w_ref[...], (((1,), (0,)), ((), ())))
    gout_ref[pl.ds(0, N), :] = h2 * dinv
    gout_ref[pl.ds(N, NPAD - N), :] = jnp.zeros((NPAD - N, D), jnp.float32)


_tc_mid = pl.pallas_call(
    _mid_body,
    out_shape=jax.ShapeDtypeStruct((NPAD, D), jnp.float32),
)


def _final_body(sp_ref, g_ref, dinv_ref, b_ref, batch_ref, o_ref):
    y = (sp_ref[0, pl.ds(0, N), :] + sp_ref[1, pl.ds(0, N), :]
         + g_ref[pl.ds(0, N), :]) * dinv_ref[...] + b_ref[...]
    seg = batch_ref[...]
    ids = lax.broadcasted_iota(jnp.int32, (N, G), 1)
    p = (seg == ids).astype(jnp.float32)
    pooled = _dot(p, y, (((0,), (0,)), ((), ())))
    cnt = _dot(p, jnp.ones((N, 1), jnp.float32), (((0,), (0,)), ((), ())))
    mean = pooled / jnp.maximum(cnt, 1.0)
    m = jnp.max(mean, axis=1, keepdims=True)
    ex = jnp.exp(mean - m)
    lse = jnp.log(jnp.sum(ex, axis=1, keepdims=True))
    o_ref[...] = mean - m - lse


_tc_final = pl.pallas_call(
    _final_body,
    out_shape=jax.ShapeDtypeStruct((G, D), jnp.float32),
)


@jax.jit
def _run(x, edge_index, batch, W1, b1, W2, b2):
    pad = (jnp.arange(EP - E, dtype=jnp.int32) % 8) + N
    srcp = jnp.concatenate([edge_index[0], pad]).reshape(NW, R, CH)
    dstp = jnp.concatenate([edge_index[1], pad]).reshape(NW, R, CH)
    zeros128 = jnp.zeros((NPAD, D), jnp.float32)
    ones128 = jnp.ones((CH, D), jnp.float32)

    degp = _sc_degree(dstp, zeros128, ones128)
    h1 = _tc_mm(x, W1)
    g1, dinv = _tc_scale(degp, h1)
    s1 = _sc_scatter(g1, srcp, dstp, zeros128)
    g2 = _tc_mid(s1, g1, dinv, b1.reshape(1, D), W2)
    s2 = _sc_scatter(g2, srcp, dstp, zeros128)
    return _tc_final(s2, g2, dinv, b2.reshape(1, D), batch.reshape(N, 1))


def kernel(x, edge_index, batch, W1, b1, W2, b2):
    return _run(x, edge_index, batch, W1, b1, W2, b2)


# speedup vs baseline: 1.1147x; 1.1147x over previous
import functools

import jax
import jax.numpy as jnp
from jax import lax
from jax.experimental import pallas as pl
from jax.experimental.pallas import tpu as pltpu
from jax.experimental.pallas import tpu_sc as plsc

N = 10000
E = 320000
D = 128
G = 64

NC = 2
NS = 16
NW = NC * NS
CH = 128
R = 80
TOTC = NW * R
EP = TOTC * CH
RC0 = 88
RC1 = 72
ROWS_PER_TILE = 632
NPAD = NS * ROWS_PER_TILE

_mesh = plsc.VectorSubcoreMesh(core_axis_name="c", subcore_axis_name="s")


def _sc_degree_body(dst_hbm, zero_hbm, ones_hbm, out_hbm, acc, di, ones, sem):
    cid = lax.axis_index("c")
    sid = lax.axis_index("s")
    base = sid * (2 * R) + cid * R
    r0 = sid * ROWS_PER_TILE
    pltpu.sync_copy(zero_hbm.at[pl.ds(r0, ROWS_PER_TILE)],
                    acc.at[pl.ds(r0, ROWS_PER_TILE)])
    pltpu.sync_copy(ones_hbm, ones)
    pltpu.async_copy(dst_hbm.at[pl.ds(base, R)], di, sem).wait()
    plsc.subcore_barrier()

    @pl.loop(0, R)
    def _(j):
        pltpu.sync_copy(ones, acc.at[di.at[j]], add=True)

    plsc.subcore_barrier()
    pltpu.sync_copy(acc.at[pl.ds(r0, ROWS_PER_TILE)],
                    out_hbm.at[cid, pl.ds(r0, ROWS_PER_TILE)])


_sc_degree = pl.kernel(
    _sc_degree_body,
    out_type=jax.ShapeDtypeStruct((NC, NPAD, D), jnp.float32),
    mesh=_mesh,
    scratch_types=[
        pltpu.VMEM_SHARED((NPAD, D), jnp.float32),
        pltpu.VMEM((R, CH), jnp.int32),
        pltpu.VMEM((CH, D), jnp.float32),
        pltpu.SemaphoreType.DMA,
    ],
)


BLK = 8


def _emit_scatter_pipeline(g_hbm, src_hbm, dst_hbm, acc,
                           sis, dis, buf0, buf1, sem0, sem1, semi,
                           base_chunk, nblk):
    pltpu.async_copy(src_hbm.at[pl.ds(base_chunk, BLK)], sis[0], semi)
    pltpu.async_copy(dst_hbm.at[pl.ds(base_chunk, BLK)], dis[0], semi)
    for b in range(nblk):
        si = sis[b % 2]
        di = dis[b % 2]
        rb = base_chunk + b * BLK
        pltpu.make_async_copy(src_hbm.at[pl.ds(rb, BLK)], si, semi).wait()
        pltpu.make_async_copy(dst_hbm.at[pl.ds(rb, BLK)], di, semi).wait()
        if b + 1 < nblk:
            pltpu.async_copy(src_hbm.at[pl.ds(rb + BLK, BLK)],
                             sis[(b + 1) % 2], semi)
            pltpu.async_copy(dst_hbm.at[pl.ds(rb + BLK, BLK)],
                             dis[(b + 1) % 2], semi)

        pltpu.async_copy(g_hbm.at[si.at[0]], buf0, sem0)

        @pl.loop(0, BLK // 2)
        def _(t):
            j = 2 * t
            pltpu.async_copy(g_hbm.at[si.at[j + 1]], buf1, sem1)
            pltpu.make_async_copy(g_hbm.at[si.at[j]], buf0, sem0).wait()
            pltpu.sync_copy(buf0, acc.at[di.at[j]], add=True)

            @pl.when(j + 2 < BLK)
            def _():
                pltpu.async_copy(g_hbm.at[si.at[j + 2]], buf0, sem0)

            pltpu.make_async_copy(g_hbm.at[si.at[j + 1]], buf1, sem1).wait()
            pltpu.sync_copy(buf1, acc.at[di.at[j + 1]], add=True)


def _sc_scatter_body(g_hbm, src_hbm, dst_hbm, zero_hbm, out_hbm,
                     acc, si0, si1, di0, di1, buf0, buf1,
                     sem0, sem1, semi):
    cid = lax.axis_index("c")
    sid = lax.axis_index("s")
    r0 = sid * ROWS_PER_TILE
    pltpu.sync_copy(zero_hbm.at[pl.ds(r0, ROWS_PER_TILE)],
                    acc.at[pl.ds(r0, ROWS_PER_TILE)])
    sis = (si0, si1)
    dis = (di0, di1)
    plsc.subcore_barrier()

    @pl.when(cid == 0)
    def _():
        _emit_scatter_pipeline(g_hbm, src_hbm, dst_hbm, acc, sis, dis,
                               buf0, buf1, sem0, sem1, semi,
                               sid * (RC0 + RC1), RC0 // BLK)

    @pl.when(cid == 1)
    def _():
        _emit_scatter_pipeline(g_hbm, src_hbm, dst_hbm, acc, sis, dis,
                               buf0, buf1, sem0, sem1, semi,
                               sid * (RC0 + RC1) + RC0, RC1 // BLK)

    plsc.subcore_barrier()
    pltpu.sync_copy(acc.at[pl.ds(r0, ROWS_PER_TILE)],
                    out_hbm.at[cid, pl.ds(r0, ROWS_PER_TILE)])


_sc_scatter = pl.kernel(
    _sc_scatter_body,
    out_type=jax.ShapeDtypeStruct((NC, NPAD, D), jnp.float32),
    mesh=_mesh,
    scratch_types=[
        pltpu.VMEM_SHARED((NPAD, D), jnp.float32),
        pltpu.VMEM((BLK, CH), jnp.int32),
        pltpu.VMEM((BLK, CH), jnp.int32),
        pltpu.VMEM((BLK, CH), jnp.int32),
        pltpu.VMEM((BLK, CH), jnp.int32),
        pltpu.VMEM((CH, D), jnp.float32),
        pltpu.VMEM((CH, D), jnp.float32),
        pltpu.SemaphoreType.DMA,
        pltpu.SemaphoreType.DMA,
        pltpu.SemaphoreType.DMA,
    ],
)


def _dot(a, b, dims):
    return lax.dot_general(a, b, dims, precision=lax.Precision.HIGHEST,
                           preferred_element_type=jnp.float32)


def _mm_body(x_ref, w_ref, o_ref):
    o_ref[...] = _dot(x_ref[...], w_ref[...], (((1,), (0,)), ((), ())))


_tc_mm = pl.pallas_call(
    _mm_body,
    out_shape=jax.ShapeDtypeStruct((N, D), jnp.float32),
)


def _scale_body(degp_ref, h_ref, g_ref, dinv_ref):
    deg = degp_ref[0, pl.ds(0, N), 0:1] + degp_ref[1, pl.ds(0, N), 0:1] + 1.0
    dinv = lax.rsqrt(deg)
    dinv_ref[...] = dinv
    g_ref[pl.ds(0, N), :] = h_ref[...] * dinv
    g_ref[pl.ds(N, NPAD - N), :] = jnp.zeros((NPAD - N, D), jnp.float32)


_tc_scale = pl.pallas_call(
    _scale_body,
    out_shape=(jax.ShapeDtypeStruct((NPAD, D), jnp.float32),
               jax.ShapeDtypeStruct((N, 1), jnp.float32)),
)


def _mid_body(sp_ref, g_ref, dinv_ref, b_ref, w_ref, gout_ref):
    dinv = dinv_ref[...]
    s = (sp_ref[0, pl.ds(0, N), :] + sp_ref[1, pl.ds(0, N), :]
         + g_ref[pl.ds(0, N), :]) * dinv + b_ref[...]
    z = jnp.maximum(s, 0.0)
    h2 = _dot(z, w_ref[...], (((1,), (0,)), ((), ())))
    gout_ref[pl.ds(0, N), :] = h2 * dinv
    gout_ref[pl.ds(N, NPAD - N), :] = jnp.zeros((NPAD - N, D), jnp.float32)


_tc_mid = pl.pallas_call(
    _mid_body,
    out_shape=jax.ShapeDtypeStruct((NPAD, D), jnp.float32),
)


def _final_body(sp_ref, g_ref, dinv_ref, b_ref, batch_ref, o_ref):
    y = (sp_ref[0, pl.ds(0, N), :] + sp_ref[1, pl.ds(0, N), :]
         + g_ref[pl.ds(0, N), :]) * dinv_ref[...] + b_ref[...]
    seg = batch_ref[...]
    ids = lax.broadcasted_iota(jnp.int32, (N, G), 1)
    p = (seg == ids).astype(jnp.float32)
    pooled = _dot(p, y, (((0,), (0,)), ((), ())))
    cnt = _dot(p, jnp.ones((N, 1), jnp.float32), (((0,), (0,)), ((), ())))
    mean = pooled / jnp.maximum(cnt, 1.0)
    m = jnp.max(mean, axis=1, keepdims=True)
    ex = jnp.exp(mean - m)
    lse = jnp.log(jnp.sum(ex, axis=1, keepdims=True))
    o_ref[...] = mean - m - lse


_tc_final = pl.pallas_call(
    _final_body,
    out_shape=jax.ShapeDtypeStruct((G, D), jnp.float32),
)


@jax.jit
def _run(x, edge_index, batch, W1, b1, W2, b2):
    pad = (jnp.arange(EP - E, dtype=jnp.int32) % 8) + N
    srcp = jnp.concatenate([edge_index[0], pad]).reshape(TOTC, CH)
    dstp = jnp.concatenate([edge_index[1], pad]).reshape(TOTC, CH)
    zeros128 = jnp.zeros((NPAD, D), jnp.float32)
    ones128 = jnp.ones((CH, D), jnp.float32)

    degp = _sc_degree(dstp, zeros128, ones128)
    h1 = _tc_mm(x, W1)
    g1, dinv = _tc_scale(degp, h1)
    s1 = _sc_scatter(g1, srcp, dstp, zeros128)
    g2 = _tc_mid(s1, g1, dinv, b1.reshape(1, D), W2)
    s2 = _sc_scatter(g2, srcp, dstp, zeros128)
    return _tc_final(s2, g2, dinv, b2.reshape(1, D), batch.reshape(N, 1))


def kernel(x, edge_index, batch, W1, b1, W2, b2):
    return _run(x, edge_index, batch, W1, b1, W2, b2)
